# Initial kernel scaffold; baseline (speedup 1.0000x reference)
import jax
import jax.numpy as jnp
from jax.experimental import pallas as pl


def kernel(output, mask, ind, target, freq_mask):
    raise NotImplementedError("write your pallas kernel here")


# speedup vs baseline: 5.0038x; 5.0038x over previous
import functools

import jax
import jax.numpy as jnp
from jax import lax
from jax.experimental import pallas as pl
from jax.experimental.pallas import tpu as pltpu
from jax.experimental.pallas import tpu_sc as plsc

B, C, H, W, K = 32, 64, 128, 128, 128
HW = H * W
NC, NS, L = 2, 16, 16
NW = NC * NS
EPW = K * C
GCH = 128
WEIGHT_ANGLE = 10.0

_mesh = plsc.VectorSubcoreMesh(core_axis_name="c", subcore_axis_name="s")


@functools.partial(
    pl.kernel,
    mesh=_mesh,
    out_type=(
        jax.ShapeDtypeStruct((NW, L), jnp.float32),
        jax.ShapeDtypeStruct((NW, L), jnp.float32),
    ),
    scratch_types=[
        pltpu.VMEM((K,), jnp.int32),
        pltpu.VMEM((K,), jnp.int32),
        pltpu.VMEM((EPW,), jnp.int32),
        pltpu.VMEM((EPW,), jnp.float32),
        pltpu.VMEM((EPW,), jnp.float32),
        pltpu.VMEM((EPW,), jnp.float32),
        pltpu.VMEM((L,), jnp.float32),
        pltpu.VMEM((L,), jnp.float32),
        pltpu.SemaphoreType.DMA,
    ],
)
def _sc_partials(out_hbm, ind_hbm, mask_hbm, tgt_hbm,
                 psum_hbm, pcnt_hbm,
                 ind_v, mask_v, idx_v, coef_v, pred_v, tgt_v,
                 psum_v, pcnt_v, sem):
    wid = lax.axis_index("s") * NC + lax.axis_index("c")

    pltpu.sync_copy(ind_hbm.at[pl.ds(wid * K, K)], ind_v)
    pltpu.sync_copy(mask_hbm.at[pl.ds(wid * K, K)], mask_v)
    pltpu.sync_copy(tgt_hbm.at[pl.ds(wid * EPW, EPW)], tgt_v)

    lanes = lax.iota(jnp.int32, L)
    base = wid * (C * HW)
    wvec = jnp.where(lanes % 2 == 0,
                     jnp.full((L,), 1.0, jnp.float32),
                     jnp.full((L,), WEIGHT_ANGLE, jnp.float32))

    def build(kb, cnt):
        vk = ind_v[pl.ds(kb * L, L)]
        mf = mask_v[pl.ds(kb * L, L)].astype(jnp.float32)
        coef16 = mf * wvec
        bvec = vk + base
        for cc in range(C):
            off = cc * K + kb * L
            idx_v[pl.ds(off, L)] = bvec + cc * HW
            coef_v[pl.ds(off, L)] = coef16
        return cnt + mf

    cnt = lax.fori_loop(0, K // L, build, jnp.zeros((L,), jnp.float32))

    def fire(j, x):
        pltpu.async_copy(out_hbm.at[idx_v.at[pl.ds(j * GCH, GCH)]],
                         pred_v.at[pl.ds(j * GCH, GCH)], sem)
        return x

    lax.fori_loop(0, EPW // GCH, fire, 0)

    def drain(j, x):
        pltpu.make_async_copy(out_hbm.at[pl.ds(0, GCH)],
                              pred_v.at[pl.ds(0, GCH)], sem).wait()
        return x

    lax.fori_loop(0, EPW // GCH, drain, 0)

    def accum(v, acc):
        d = pred_v[pl.ds(v * L, L)] - tgt_v[pl.ds(v * L, L)]
        return acc + coef_v[pl.ds(v * L, L)] * jnp.abs(d)

    acc = lax.fori_loop(0, EPW // L, accum, jnp.zeros((L,), jnp.float32))

    psum_v[...] = acc
    pcnt_v[...] = cnt
    pltpu.sync_copy(psum_v, psum_hbm.at[wid])
    pltpu.sync_copy(pcnt_v, pcnt_hbm.at[wid])


def _finish_body(ps_ref, pc_ref, o_ref):
    total = jnp.sum(ps_ref[...])
    count = jnp.sum(pc_ref[...])
    o_ref[...] = jnp.broadcast_to(total / (count * float(C) + 1e-4), (1, 1))


_finish = pl.pallas_call(
    _finish_body,
    out_shape=jax.ShapeDtypeStruct((1, 1), jnp.float32),
)


def kernel(output, mask, ind, target, freq_mask):
    del freq_mask
    psum, pcnt = _sc_partials(
        output.reshape(-1),
        ind.reshape(-1).astype(jnp.int32),
        mask.reshape(-1).astype(jnp.int32),
        target.transpose(0, 2, 1).reshape(-1),
    )
    return _finish(psum, pcnt)[0, 0]
